# baseline (speedup 1.0000x reference)
import jax
import jax.numpy as jnp
from jax import lax
from jax.experimental import pallas as pl
from jax.experimental.pallas import tpu as pltpu
from jax.experimental.pallas import tpu_sc as plsc

E = 1600000
D = 32
V = 1024
TBL = 3 * V * D
C = 80
UNROLL = 1

_info = plsc.get_sparse_core_info()
NC, NS = _info.num_cores, _info.num_subcores
NW = NC * NS
PER_W = E // NW
NCHUNK = PER_W // C


def _body(i0_hbm, i1_hbm, i2_hbm, w_hbm, out_hbm,
          wv, ova, ovb, ia0, ia1, ia2, ib0, ib1, ib2, isem, osem):
    w = lax.axis_index("s") * NC + lax.axis_index("c")
    w_base = w * PER_W
    idx_srcs = (i0_hbm, i1_hbm, i2_hbm)
    ivs = ((ia0, ia1, ia2), (ib0, ib1, ib2))
    ovs = (ova, ovb)

    pltpu.sync_copy(w_hbm, wv)

    def chunk_base(k):
        return w_base + jnp.minimum(k, NCHUNK - 1) * C

    def load_idx(b, k):
        base = chunk_base(k)
        for t in range(3):
            pltpu.async_copy(idx_srcs[t].at[pl.ds(base, C)], ivs[b][t],
                             isem.at[b])

    def wait_idx(b):
        for t in range(3):
            pltpu.make_async_copy(idx_srcs[t].at[pl.ds(w_base, C)],
                                  ivs[b][t], isem.at[b]).wait()

    def compute_store(b, k, first):
        ov = ovs[b]
        @pl.when(jnp.logical_not(first))
        def _():
            pltpu.make_async_copy(ov, out_hbm.at[pl.ds(w_base, C)],
                                  osem.at[b]).wait()

        @plsc.parallel_loop(0, C, 16, unroll=UNROLL)
        def _(g):
            v0 = ivs[b][0][pl.ds(g, 16)] * D
            v1 = ivs[b][1][pl.ds(g, 16)] * D + V * D
            v2 = ivs[b][2][pl.ds(g, 16)] * D + 2 * V * D
            for lane in range(16):
                a0, a1, a2 = v0[lane], v1[lane], v2[lane]
                for h in range(D // 16):
                    ov[g + lane, pl.ds(h * 16, 16)] = (
                        wv[pl.ds(a0 + h * 16, 16)]
                        + wv[pl.ds(a1 + h * 16, 16)]
                        + wv[pl.ds(a2 + h * 16, 16)])

        pltpu.async_copy(ov, out_hbm.at[pl.ds(chunk_base(k), C)], osem.at[b])

    load_idx(0, 0)
    load_idx(1, 1)

    def pair_body(p, carry):
        k = p * 2
        wait_idx(0)
        compute_store(0, k, p == 0)
        load_idx(0, k + 2)
        wait_idx(1)
        compute_store(1, k + 1, p == 0)
        load_idx(1, k + 3)
        return carry

    lax.fori_loop(0, NCHUNK // 2, pair_body, 0)

    wait_idx(0)
    compute_store(0, NCHUNK - 1, False)
    wait_idx(1)
    for b in range(2):
        pltpu.make_async_copy(ovs[b], out_hbm.at[pl.ds(w_base, C)],
                              osem.at[b]).wait()


def kernel(edge_attr, W0, W1, W2):
    idx0 = edge_attr[:, 0]
    idx1 = edge_attr[:, 1]
    idx2 = edge_attr[:, 2]
    w_flat = jnp.concatenate([W0, W1, W2], axis=0).reshape(-1)
    run = pl.kernel(
        _body,
        out_type=jax.ShapeDtypeStruct((E, D), jnp.float32),
        mesh=plsc.VectorSubcoreMesh(core_axis_name="c", subcore_axis_name="s"),
        compiler_params=pltpu.CompilerParams(use_tc_tiling_on_sc=False),
        scratch_types=[
            pltpu.VMEM((TBL,), jnp.float32),
            pltpu.VMEM((C, D), jnp.float32),
            pltpu.VMEM((C, D), jnp.float32),
            pltpu.VMEM((C,), jnp.int32),
            pltpu.VMEM((C,), jnp.int32),
            pltpu.VMEM((C,), jnp.int32),
            pltpu.VMEM((C,), jnp.int32),
            pltpu.VMEM((C,), jnp.int32),
            pltpu.VMEM((C,), jnp.int32),
            pltpu.SemaphoreType.DMA((2,)),
            pltpu.SemaphoreType.DMA((2,)),
        ],
    )
    return run(idx0, idx1, idx2, w_flat)


# speedup vs baseline: 1.0004x; 1.0004x over previous
import jax
import jax.numpy as jnp
from jax import lax
from jax.experimental import pallas as pl
from jax.experimental.pallas import tpu as pltpu
from jax.experimental.pallas import tpu_sc as plsc

E = 1600000
D = 32
V = 1024
TBL = 3 * V * D
C = 80
UNROLL = 1

_info = plsc.get_sparse_core_info()
NC, NS = _info.num_cores, _info.num_subcores
NW = NC * NS
PER_W = E // NW
NCHUNK = PER_W // C


def _body(i0_hbm, i1_hbm, i2_hbm, w_hbm, out_hbm,
          wv, ova, ovb, ia0, ia1, ia2, ib0, ib1, ib2, isem, osem):
    w = lax.axis_index("s") * NC + lax.axis_index("c")
    w_base = w * PER_W
    idx_srcs = (i0_hbm, i1_hbm, i2_hbm)
    ivs = ((ia0, ia1, ia2), (ib0, ib1, ib2))
    ovs = (ova, ovb)

    pltpu.sync_copy(w_hbm, wv)

    def chunk_base(k):
        return w_base + jnp.minimum(k, NCHUNK - 1) * C

    def load_idx(b, k):
        base = chunk_base(k)
        for t in range(3):
            pltpu.async_copy(idx_srcs[t].at[pl.ds(base, C)], ivs[b][t],
                             isem.at[b])

    def wait_idx(b):
        for t in range(3):
            pltpu.make_async_copy(idx_srcs[t].at[pl.ds(w_base, C)],
                                  ivs[b][t], isem.at[b]).wait()

    def compute_store(b, k, first):
        ov = ovs[b]
        @pl.when(jnp.logical_not(first))
        def _():
            pltpu.make_async_copy(ov, out_hbm.at[pl.ds(w_base * D, C * D)],
                                  osem.at[b]).wait()

        @plsc.parallel_loop(0, C, 16, unroll=UNROLL)
        def _(g):
            v0 = ivs[b][0][pl.ds(g, 16)] * D
            v1 = ivs[b][1][pl.ds(g, 16)] * D + V * D
            v2 = ivs[b][2][pl.ds(g, 16)] * D + 2 * V * D
            for lane in range(16):
                a0, a1, a2 = v0[lane], v1[lane], v2[lane]
                for h in range(D // 16):
                    ov[pl.ds((g + lane) * D + h * 16, 16)] = (
                        wv[pl.ds(a0 + h * 16, 16)]
                        + wv[pl.ds(a1 + h * 16, 16)]
                        + wv[pl.ds(a2 + h * 16, 16)])

        pltpu.async_copy(ov, out_hbm.at[pl.ds(chunk_base(k) * D, C * D)],
                         osem.at[b])

    load_idx(0, 0)
    load_idx(1, 1)

    def pair_body(p, carry):
        k = p * 2
        wait_idx(0)
        compute_store(0, k, p == 0)
        load_idx(0, k + 2)
        wait_idx(1)
        compute_store(1, k + 1, p == 0)
        load_idx(1, k + 3)
        return carry

    lax.fori_loop(0, NCHUNK // 2, pair_body, 0)

    wait_idx(0)
    compute_store(0, NCHUNK - 1, False)
    wait_idx(1)
    for b in range(2):
        pltpu.make_async_copy(ovs[b], out_hbm.at[pl.ds(w_base * D, C * D)],
                              osem.at[b]).wait()


def kernel(edge_attr, W0, W1, W2):
    idx0 = edge_attr[:, 0]
    idx1 = edge_attr[:, 1]
    idx2 = edge_attr[:, 2]
    w_flat = jnp.concatenate([W0, W1, W2], axis=0).reshape(-1)
    run = pl.kernel(
        _body,
        out_type=jax.ShapeDtypeStruct((E * D,), jnp.float32),
        mesh=plsc.VectorSubcoreMesh(core_axis_name="c", subcore_axis_name="s"),
        compiler_params=pltpu.CompilerParams(use_tc_tiling_on_sc=True),
        scratch_types=[
            pltpu.VMEM((TBL,), jnp.float32),
            pltpu.VMEM((C * D,), jnp.float32),
            pltpu.VMEM((C * D,), jnp.float32),
            pltpu.VMEM((C,), jnp.int32),
            pltpu.VMEM((C,), jnp.int32),
            pltpu.VMEM((C,), jnp.int32),
            pltpu.VMEM((C,), jnp.int32),
            pltpu.VMEM((C,), jnp.int32),
            pltpu.VMEM((C,), jnp.int32),
            pltpu.SemaphoreType.DMA((2,)),
            pltpu.SemaphoreType.DMA((2,)),
        ],
    )
    return run(idx0, idx1, idx2, w_flat).reshape(E, D)


# speedup vs baseline: 1.2413x; 1.2408x over previous
import jax
import jax.numpy as jnp
from jax import lax
from jax.experimental import pallas as pl
from jax.experimental.pallas import tpu as pltpu
from jax.experimental.pallas import tpu_sc as plsc

E = 1600000
D = 32
V = 1024
TBL = 3 * V * D
C = 80
UNROLL = 1
S = 5
SL = E // S

_info = plsc.get_sparse_core_info()
NC, NS = _info.num_cores, _info.num_subcores
NW = NC * NS
PER_W = SL // NW
NCHUNK = PER_W // C


def _body(i0_hbm, i1_hbm, i2_hbm, w_hbm, out_hbm,
          wv, ova, ovb, ia0, ia1, ia2, ib0, ib1, ib2, isem, osem):
    w = lax.axis_index("s") * NC + lax.axis_index("c")
    w_base = w * PER_W
    idx_srcs = (i0_hbm, i1_hbm, i2_hbm)
    ivs = ((ia0, ia1, ia2), (ib0, ib1, ib2))
    ovs = (ova, ovb)

    pltpu.sync_copy(w_hbm, wv)

    def chunk_base(k):
        return w_base + jnp.minimum(k, NCHUNK - 1) * C

    def load_idx(b, k):
        base = chunk_base(k)
        for t in range(3):
            pltpu.async_copy(idx_srcs[t].at[pl.ds(base, C)], ivs[b][t],
                             isem.at[b])

    def wait_idx(b):
        for t in range(3):
            pltpu.make_async_copy(idx_srcs[t].at[pl.ds(w_base, C)],
                                  ivs[b][t], isem.at[b]).wait()

    def compute_store(b, k, first):
        ov = ovs[b]
        @pl.when(jnp.logical_not(first))
        def _():
            pltpu.make_async_copy(ov, out_hbm.at[pl.ds(w_base, C)],
                                  osem.at[b]).wait()

        @plsc.parallel_loop(0, C, 16, unroll=UNROLL)
        def _(g):
            v0 = ivs[b][0][pl.ds(g, 16)] * D
            v1 = ivs[b][1][pl.ds(g, 16)] * D + V * D
            v2 = ivs[b][2][pl.ds(g, 16)] * D + 2 * V * D
            for lane in range(16):
                a0, a1, a2 = v0[lane], v1[lane], v2[lane]
                for h in range(D // 16):
                    ov[g + lane, pl.ds(h * 16, 16)] = (
                        wv[pl.ds(a0 + h * 16, 16)]
                        + wv[pl.ds(a1 + h * 16, 16)]
                        + wv[pl.ds(a2 + h * 16, 16)])

        pltpu.async_copy(ov, out_hbm.at[pl.ds(chunk_base(k), C)],
                         osem.at[b])

    load_idx(0, 0)
    load_idx(1, 1)

    def pair_body(p, carry):
        k = p * 2
        wait_idx(0)
        compute_store(0, k, p == 0)
        load_idx(0, k + 2)
        wait_idx(1)
        compute_store(1, k + 1, p == 0)
        load_idx(1, k + 3)
        return carry

    lax.fori_loop(0, NCHUNK // 2, pair_body, 0)

    wait_idx(0)
    compute_store(0, NCHUNK - 1, False)
    wait_idx(1)
    for b in range(2):
        pltpu.make_async_copy(ovs[b], out_hbm.at[pl.ds(w_base, C)],
                              osem.at[b]).wait()


def kernel(edge_attr, W0, W1, W2):
    idx0 = edge_attr[:, 0]
    idx1 = edge_attr[:, 1]
    idx2 = edge_attr[:, 2]
    w_flat = jnp.concatenate([W0, W1, W2], axis=0).reshape(-1)
    run = pl.kernel(
        _body,
        out_type=jax.ShapeDtypeStruct((SL, D), jnp.float32),
        mesh=plsc.VectorSubcoreMesh(core_axis_name="c", subcore_axis_name="s"),
        compiler_params=pltpu.CompilerParams(use_tc_tiling_on_sc=True),
        scratch_types=[
            pltpu.VMEM((TBL,), jnp.float32),
            pltpu.VMEM((C, D), jnp.float32),
            pltpu.VMEM((C, D), jnp.float32),
            pltpu.VMEM((C,), jnp.int32),
            pltpu.VMEM((C,), jnp.int32),
            pltpu.VMEM((C,), jnp.int32),
            pltpu.VMEM((C,), jnp.int32),
            pltpu.VMEM((C,), jnp.int32),
            pltpu.VMEM((C,), jnp.int32),
            pltpu.SemaphoreType.DMA((2,)),
            pltpu.SemaphoreType.DMA((2,)),
        ],
    )
    outs = [run(idx0[s * SL:(s + 1) * SL], idx1[s * SL:(s + 1) * SL],
                idx2[s * SL:(s + 1) * SL], w_flat) for s in range(S)]
    return jnp.concatenate(outs, axis=0)


# speedup vs baseline: 1.3070x; 1.0530x over previous
import jax
import jax.numpy as jnp
from jax import lax
from jax.experimental import pallas as pl
from jax.experimental.pallas import tpu as pltpu
from jax.experimental.pallas import tpu_sc as plsc

E = 1600000
D = 32
V = 1024
TBL = 3 * V * D
C = 80
UNROLL = 1

_info = plsc.get_sparse_core_info()
NC, NS = _info.num_cores, _info.num_subcores
NW = NC * NS
PER_W = E // NW
NCHUNK = PER_W // C


def _body(i0_hbm, i1_hbm, i2_hbm, w_hbm, out_hbm,
          wv, ova, ovb, ia0, ia1, ia2, ib0, ib1, ib2, isem, osem):
    w = lax.axis_index("s") * NC + lax.axis_index("c")
    w_base = w * PER_W
    idx_srcs = (i0_hbm, i1_hbm, i2_hbm)
    ivs = ((ia0, ia1, ia2), (ib0, ib1, ib2))
    ovs = (ova, ovb)

    pltpu.sync_copy(w_hbm, wv)

    def chunk_base(k):
        return w_base + jnp.minimum(k, NCHUNK - 1) * C

    def load_idx(b, k):
        base = chunk_base(k)
        for t in range(3):
            pltpu.async_copy(idx_srcs[t].at[pl.ds(base, C)], ivs[b][t],
                             isem.at[b])

    def wait_idx(b):
        for t in range(3):
            pltpu.make_async_copy(idx_srcs[t].at[pl.ds(w_base, C)],
                                  ivs[b][t], isem.at[b]).wait()

    def compute_store(b, k, first):
        ov = ovs[b]
        @pl.when(jnp.logical_not(first))
        def _():
            pltpu.make_async_copy(ov, out_hbm.at[pl.ds(w_base, C)],
                                  osem.at[b]).wait()

        @plsc.parallel_loop(0, C, 16, unroll=UNROLL)
        def _(g):
            v0 = ivs[b][0][pl.ds(g, 16)] * D
            v1 = ivs[b][1][pl.ds(g, 16)] * D + V * D
            v2 = ivs[b][2][pl.ds(g, 16)] * D + 2 * V * D
            for lane in range(16):
                a0, a1, a2 = v0[lane], v1[lane], v2[lane]
                for h in range(D // 16):
                    ov[g + lane, pl.ds(h * 16, 16)] = (
                        wv[pl.ds(a0 + h * 16, 16)]
                        + wv[pl.ds(a1 + h * 16, 16)]
                        + wv[pl.ds(a2 + h * 16, 16)])

        pltpu.async_copy(ov, out_hbm.at[pl.ds(chunk_base(k), C)],
                         osem.at[b])

    load_idx(0, 0)
    load_idx(1, 1)

    def pair_body(p, carry):
        k = p * 2
        wait_idx(0)
        compute_store(0, k, p == 0)
        load_idx(0, k + 2)
        wait_idx(1)
        compute_store(1, k + 1, p == 0)
        load_idx(1, k + 3)
        return carry

    lax.fori_loop(0, NCHUNK // 2, pair_body, 0)

    wait_idx(0)
    compute_store(0, NCHUNK - 1, False)
    wait_idx(1)
    for b in range(2):
        pltpu.make_async_copy(ovs[b], out_hbm.at[pl.ds(w_base, C)],
                              osem.at[b]).wait()


def kernel(edge_attr, W0, W1, W2):
    idx0 = edge_attr[:, 0]
    idx1 = edge_attr[:, 1]
    idx2 = edge_attr[:, 2]
    w_flat = jnp.concatenate([W0, W1, W2], axis=0).reshape(-1)
    run = pl.kernel(
        _body,
        out_type=jax.ShapeDtypeStruct((E, D), jnp.float32),
        mesh=plsc.VectorSubcoreMesh(core_axis_name="c", subcore_axis_name="s"),
        compiler_params=pltpu.CompilerParams(use_tc_tiling_on_sc=True),
        scratch_types=[
            pltpu.VMEM((TBL,), jnp.float32),
            pltpu.VMEM((C, D), jnp.float32),
            pltpu.VMEM((C, D), jnp.float32),
            pltpu.VMEM((C,), jnp.int32),
            pltpu.VMEM((C,), jnp.int32),
            pltpu.VMEM((C,), jnp.int32),
            pltpu.VMEM((C,), jnp.int32),
            pltpu.VMEM((C,), jnp.int32),
            pltpu.VMEM((C,), jnp.int32),
            pltpu.SemaphoreType.DMA((2,)),
            pltpu.SemaphoreType.DMA((2,)),
        ],
    )
    return run(idx0, idx1, idx2, w_flat)
